# baseline (speedup 1.0000x reference)
import functools

import jax
import jax.numpy as jnp
from jax.experimental import pallas as pl
from jax.experimental.pallas import tpu as pltpu

_RATIO = 0.2


def _msiw_body(x_ref, out_ref, cnt_ref, ssum_ref, *, nsteps, c, np_total, n_batch):
    i = pl.program_id(0)

    @pl.when(i == 0)
    def _init():
        cnt_ref[...] = jnp.zeros_like(cnt_ref)
        ssum_ref[...] = jnp.zeros_like(ssum_ref)

    m = x_ref[0, 0]
    for ci in range(1, c):
        m = jnp.maximum(m, x_ref[0, ci])

    z = jnp.zeros_like(m)
    s2 = jnp.zeros_like(m)
    for ci in range(c):
        e = jnp.exp(x_ref[0, ci] - m)
        z += e
        s2 += e * e
    s = s2 / (z * z)

    taken = jnp.zeros(m.shape, dtype=jnp.bool_)
    for ci in range(c):
        eq = x_ref[0, ci] == m
        hit = jnp.logical_and(eq, jnp.logical_not(taken))
        taken = jnp.logical_or(taken, eq)
        cnt_ref[ci] += jnp.where(hit, 1.0, 0.0)
        ssum_ref[ci] += jnp.where(hit, s, 0.0)

    @pl.when(i == nsteps - 1)
    def _finish():
        cnt_t = jnp.sum(cnt_ref[...], axis=(1, 2), keepdims=True)[:, 0, :]
        s_t = jnp.sum(ssum_ref[...], axis=(1, 2), keepdims=True)[:, 0, :]
        np_pow = float(np_total) ** (1.0 - _RATIO)
        pos = cnt_t > 0.0
        den_raw = jnp.exp(_RATIO * jnp.log(jnp.where(pos, cnt_t, 1.0))) * np_pow
        den = jnp.maximum(jnp.where(pos, den_raw, 0.0), 1.0)
        total = jnp.sum(s_t / den, axis=0, keepdims=True)
        out_ref[...] = -total / (n_batch * c)


def kernel(nw_out):
    n, c, h, w = nw_out.shape
    bh = 8
    nh = h // bh
    nsteps = n * nh
    np_total = n * h * w

    body = functools.partial(
        _msiw_body, nsteps=nsteps, c=c, np_total=np_total, n_batch=n
    )
    out = pl.pallas_call(
        body,
        grid=(nsteps,),
        in_specs=[
            pl.BlockSpec((1, c, bh, w), lambda i: (i // nh, 0, i % nh, 0)),
        ],
        out_specs=pl.BlockSpec((1, 1), lambda i: (0, 0)),
        out_shape=jax.ShapeDtypeStruct((1, 1), jnp.float32),
        scratch_shapes=[
            pltpu.VMEM((c, bh, w), jnp.float32),
            pltpu.VMEM((c, bh, w), jnp.float32),
        ],
        compiler_params=pltpu.CompilerParams(
            dimension_semantics=("arbitrary",),
        ),
    )(nw_out)
    return out[0, 0]


# speedup vs baseline: 3.4007x; 3.4007x over previous
import functools

import jax
import jax.numpy as jnp
from jax.experimental import pallas as pl
from jax.experimental.pallas import tpu as pltpu

_RATIO = 0.2


def _msiw_body(x_ref, out_ref, cnt_ref, ssum_ref, *, nsteps, c, np_total, n_batch):
    i = pl.program_id(0)

    @pl.when(i == 0)
    def _init():
        cnt_ref[...] = jnp.zeros_like(cnt_ref)
        ssum_ref[...] = jnp.zeros_like(ssum_ref)

    bh = x_ref.shape[2]
    for r in range(0, bh, 8):
        m = x_ref[0, 0, r : r + 8]
        for ci in range(1, c):
            m = jnp.maximum(m, x_ref[0, ci, r : r + 8])

        z = jnp.zeros_like(m)
        s2 = jnp.zeros_like(m)
        for ci in range(c):
            e = jnp.exp(x_ref[0, ci, r : r + 8] - m)
            z += e
            s2 += e * e
        s = s2 / (z * z)

        taken = jnp.zeros(m.shape, dtype=jnp.bool_)
        for ci in range(c):
            eq = x_ref[0, ci, r : r + 8] == m
            hit = jnp.logical_and(eq, jnp.logical_not(taken))
            taken = jnp.logical_or(taken, eq)
            cnt_ref[ci] += jnp.where(hit, 1.0, 0.0)
            ssum_ref[ci] += jnp.where(hit, s, 0.0)

    @pl.when(i == nsteps - 1)
    def _finish():
        cnt_t = jnp.sum(cnt_ref[...], axis=(1, 2), keepdims=True)[:, 0, :]
        s_t = jnp.sum(ssum_ref[...], axis=(1, 2), keepdims=True)[:, 0, :]
        np_pow = float(np_total) ** (1.0 - _RATIO)
        pos = cnt_t > 0.0
        den_raw = jnp.exp(_RATIO * jnp.log(jnp.where(pos, cnt_t, 1.0))) * np_pow
        den = jnp.maximum(jnp.where(pos, den_raw, 0.0), 1.0)
        total = jnp.sum(s_t / den, axis=0, keepdims=True)
        out_ref[...] = -total / (n_batch * c)


def kernel(nw_out):
    n, c, h, w = nw_out.shape
    bh = 64
    nh = h // bh
    nsteps = n * nh
    np_total = n * h * w

    body = functools.partial(
        _msiw_body, nsteps=nsteps, c=c, np_total=np_total, n_batch=n
    )
    out = pl.pallas_call(
        body,
        grid=(nsteps,),
        in_specs=[
            pl.BlockSpec((1, c, bh, w), lambda i: (i // nh, 0, i % nh, 0)),
        ],
        out_specs=pl.BlockSpec((1, 1), lambda i: (0, 0)),
        out_shape=jax.ShapeDtypeStruct((1, 1), jnp.float32),
        scratch_shapes=[
            pltpu.VMEM((c, 8, w), jnp.float32),
            pltpu.VMEM((c, 8, w), jnp.float32),
        ],
        compiler_params=pltpu.CompilerParams(
            dimension_semantics=("arbitrary",),
        ),
    )(nw_out)
    return out[0, 0]
